# baseline (speedup 1.0000x reference)
import functools

import jax
import jax.numpy as jnp
from jax import lax
from jax.experimental import pallas as pl
from jax.experimental.pallas import tpu as pltpu
from jax.experimental.pallas import tpu_sc as plsc

N = 100000
D = 128

NC = 2
NS = 16
NW = NC * NS

CHUNK = 128
NP = 102400
ROWS_PER_W = NP // NW
NCHUNK = ROWS_PER_W // CHUNK


def _sc_gather(x, idx3):
    mesh = plsc.VectorSubcoreMesh(
        core_axis_name="c", subcore_axis_name="s", num_cores=NC, num_subcores=NS
    )

    NBUF = 4

    @functools.partial(
        pl.kernel,
        out_type=jax.ShapeDtypeStruct((NP, D), jnp.float32),
        mesh=mesh,
        scratch_types=[
            pltpu.VMEM((NCHUNK, CHUNK), jnp.int32),
            pltpu.VMEM((NBUF, CHUNK, D), jnp.float32),
            pltpu.SemaphoreType.DMA((NBUF,)),
            pltpu.SemaphoreType.DMA((NBUF,)),
        ],
    )
    def gather_kernel(x_hbm, idx_hbm, g_hbm, idx_v, buf, gsem, wsem):
        wid = lax.axis_index("s") * NC + lax.axis_index("c")
        pltpu.sync_copy(idx_hbm.at[wid], idx_v)
        base = wid * ROWS_PER_W

        for j in range(NBUF - 1):
            pltpu.async_copy(x_hbm.at[idx_v.at[j]], buf.at[j], gsem.at[j])

        def chunk_body(i, carry):
            slot = lax.rem(i, NBUF)
            nxt = i + NBUF - 1
            nslot = lax.rem(nxt, NBUF)

            @pl.when(nxt < NCHUNK)
            def _():
                @pl.when(i >= 1)
                def _():
                    pltpu.make_async_copy(
                        buf.at[nslot],
                        g_hbm.at[pl.ds(base + (i - 1) * CHUNK, CHUNK)],
                        wsem.at[nslot],
                    ).wait()

                pltpu.async_copy(x_hbm.at[idx_v.at[nxt]], buf.at[nslot], gsem.at[nslot])

            pltpu.make_async_copy(
                x_hbm.at[idx_v.at[i]], buf.at[slot], gsem.at[slot]
            ).wait()
            pltpu.async_copy(
                buf.at[slot], g_hbm.at[pl.ds(base + i * CHUNK, CHUNK)], wsem.at[slot]
            )
            return carry

        lax.fori_loop(0, NCHUNK, chunk_body, 0, unroll=False)

        for i in range(NCHUNK - NBUF, NCHUNK):
            s = i % NBUF
            pltpu.make_async_copy(
                buf.at[s], g_hbm.at[pl.ds(base + i * CHUNK, CHUNK)], wsem.at[s]
            ).wait()

    return gather_kernel(x, idx3)


def _tc_linear(g, x, wa, wb, b2):
    R = 2000
    nblk = N // R

    def body(g_ref, x_ref, wa_ref, wb_ref, b_ref, o_ref):
        acc = jnp.dot(g_ref[...], wa_ref[...], preferred_element_type=jnp.float32)
        acc = acc + jnp.dot(x_ref[...], wb_ref[...], preferred_element_type=jnp.float32)
        o_ref[...] = acc + b_ref[...]

    return pl.pallas_call(
        body,
        grid=(nblk,),
        in_specs=[
            pl.BlockSpec((R, D), lambda i: (i, 0)),
            pl.BlockSpec((R, D), lambda i: (i, 0)),
            pl.BlockSpec((D, D), lambda i: (0, 0)),
            pl.BlockSpec((D, D), lambda i: (0, 0)),
            pl.BlockSpec((1, D), lambda i: (0, 0)),
        ],
        out_specs=pl.BlockSpec((R, D), lambda i: (i, 0)),
        out_shape=jax.ShapeDtypeStruct((N, D), jnp.float32),
        compiler_params=pltpu.CompilerParams(
            dimension_semantics=("arbitrary",),
        ),
    )(g, x, wa, wb, b2)


def kernel(x, idx, W2, b2):
    idx_pad = jnp.concatenate([idx, jnp.zeros((NP - N,), jnp.int32)])
    idx3 = idx_pad.reshape(NW, NCHUNK, CHUNK)
    g = _sc_gather(x, idx3)
    wa = W2[:D]
    wb = W2[D:]
    return _tc_linear(g, x, wa, wb, b2.reshape(1, D))


# speedup vs baseline: 1.0074x; 1.0074x over previous
import functools

import jax
import jax.numpy as jnp
from jax import lax
from jax.experimental import pallas as pl
from jax.experimental.pallas import tpu as pltpu
from jax.experimental.pallas import tpu_sc as plsc

N = 100000
D = 128

NC = 2
NS = 16
NW = NC * NS

CHUNK = 128
NP = 102400
ROWS_PER_W = NP // NW
NCHUNK = ROWS_PER_W // CHUNK
NBUF = 6


def _sc_gather(x, idx3):
    mesh = plsc.VectorSubcoreMesh(
        core_axis_name="c", subcore_axis_name="s", num_cores=NC, num_subcores=NS
    )

    @functools.partial(
        pl.kernel,
        out_type=jax.ShapeDtypeStruct((NP, D), jnp.float32),
        mesh=mesh,
        scratch_types=[
            pltpu.VMEM((NCHUNK, CHUNK), jnp.int32),
            pltpu.VMEM((NBUF, CHUNK, D), jnp.float32),
            pltpu.SemaphoreType.DMA((NBUF,)),
            pltpu.SemaphoreType.DMA((NBUF,)),
        ],
    )
    def gather_kernel(x_hbm, idx_hbm, g_hbm, idx_v, buf, gsem, wsem):
        wid = lax.axis_index("s") * NC + lax.axis_index("c")
        pltpu.sync_copy(idx_hbm.at[wid], idx_v)
        base = wid * ROWS_PER_W

        for j in range(NBUF - 1):
            pltpu.async_copy(x_hbm.at[idx_v.at[j]], buf.at[j], gsem.at[j])

        def chunk_body(i, carry):
            slot = lax.rem(i, NBUF)
            nxt = i + NBUF - 1
            nslot = lax.rem(nxt, NBUF)

            @pl.when(nxt < NCHUNK)
            def _():
                @pl.when(i >= 1)
                def _():
                    pltpu.make_async_copy(
                        buf.at[nslot],
                        g_hbm.at[pl.ds(base + (i - 1) * CHUNK, CHUNK)],
                        wsem.at[nslot],
                    ).wait()

                pltpu.async_copy(x_hbm.at[idx_v.at[nxt]], buf.at[nslot], gsem.at[nslot])

            pltpu.make_async_copy(
                x_hbm.at[idx_v.at[i]], buf.at[slot], gsem.at[slot]
            ).wait()
            pltpu.async_copy(
                buf.at[slot], g_hbm.at[pl.ds(base + i * CHUNK, CHUNK)], wsem.at[slot]
            )
            return carry

        lax.fori_loop(0, NCHUNK, chunk_body, 0, unroll=False)

        for i in range(NCHUNK - NBUF, NCHUNK):
            s = i % NBUF
            pltpu.make_async_copy(
                buf.at[s], g_hbm.at[pl.ds(base + i * CHUNK, CHUNK)], wsem.at[s]
            ).wait()

    return gather_kernel(x, idx3)


def _tc_linear(g, x, wa, wb, b2):
    R = 2000
    nblk = N // R

    def body(g_ref, x_ref, wa_ref, wb_ref, b_ref, o_ref):
        acc = jnp.dot(g_ref[...], wa_ref[...], preferred_element_type=jnp.float32)
        acc = acc + jnp.dot(x_ref[...], wb_ref[...], preferred_element_type=jnp.float32)
        o_ref[...] = acc + b_ref[...]

    return pl.pallas_call(
        body,
        grid=(nblk,),
        in_specs=[
            pl.BlockSpec((R, D), lambda i: (i, 0)),
            pl.BlockSpec((R, D), lambda i: (i, 0)),
            pl.BlockSpec((D, D), lambda i: (0, 0)),
            pl.BlockSpec((D, D), lambda i: (0, 0)),
            pl.BlockSpec((1, D), lambda i: (0, 0)),
        ],
        out_specs=pl.BlockSpec((R, D), lambda i: (i, 0)),
        out_shape=jax.ShapeDtypeStruct((N, D), jnp.float32),
        compiler_params=pltpu.CompilerParams(
            dimension_semantics=("arbitrary",),
        ),
    )(g, x, wa, wb, b2)


def kernel(x, idx, W2, b2):
    idx_pad = jnp.concatenate([idx, jnp.zeros((NP - N,), jnp.int32)])
    idx3 = idx_pad.reshape(NW, NCHUNK, CHUNK)
    g = _sc_gather(x, idx3)
    wa = W2[:D]
    wb = W2[D:]
    return _tc_linear(g, x, wa, wb, b2.reshape(1, D))


# speedup vs baseline: 1.0101x; 1.0026x over previous
import functools

import jax
import jax.numpy as jnp
from jax import lax
from jax.experimental import pallas as pl
from jax.experimental.pallas import tpu as pltpu
from jax.experimental.pallas import tpu_sc as plsc

N = 100000
D = 128

NC = 1
NS = 16
NW = NC * NS

CHUNK = 128
NP = 102400
ROWS_PER_W = NP // NW
NCHUNK = ROWS_PER_W // CHUNK
NBUF = 6


def _sc_gather(x, idx3):
    mesh = plsc.VectorSubcoreMesh(
        core_axis_name="c", subcore_axis_name="s", num_cores=NC, num_subcores=NS
    )

    @functools.partial(
        pl.kernel,
        out_type=jax.ShapeDtypeStruct((NP, D), jnp.float32),
        mesh=mesh,
        scratch_types=[
            pltpu.VMEM((NCHUNK, CHUNK), jnp.int32),
            pltpu.VMEM((NBUF, CHUNK, D), jnp.float32),
            pltpu.SemaphoreType.DMA((NBUF,)),
            pltpu.SemaphoreType.DMA((NBUF,)),
        ],
    )
    def gather_kernel(x_hbm, idx_hbm, g_hbm, idx_v, buf, gsem, wsem):
        wid = lax.axis_index("s") * NC + lax.axis_index("c")
        pltpu.sync_copy(idx_hbm.at[wid], idx_v)
        base = wid * ROWS_PER_W

        for j in range(NBUF - 1):
            pltpu.async_copy(x_hbm.at[idx_v.at[j]], buf.at[j], gsem.at[j])

        def chunk_body(i, carry):
            slot = lax.rem(i, NBUF)
            nxt = i + NBUF - 1
            nslot = lax.rem(nxt, NBUF)

            @pl.when(nxt < NCHUNK)
            def _():
                @pl.when(i >= 1)
                def _():
                    pltpu.make_async_copy(
                        buf.at[nslot],
                        g_hbm.at[pl.ds(base + (i - 1) * CHUNK, CHUNK)],
                        wsem.at[nslot],
                    ).wait()

                pltpu.async_copy(x_hbm.at[idx_v.at[nxt]], buf.at[nslot], gsem.at[nslot])

            pltpu.make_async_copy(
                x_hbm.at[idx_v.at[i]], buf.at[slot], gsem.at[slot]
            ).wait()
            pltpu.async_copy(
                buf.at[slot], g_hbm.at[pl.ds(base + i * CHUNK, CHUNK)], wsem.at[slot]
            )
            return carry

        lax.fori_loop(0, NCHUNK, chunk_body, 0, unroll=False)

        for i in range(NCHUNK - NBUF, NCHUNK):
            s = i % NBUF
            pltpu.make_async_copy(
                buf.at[s], g_hbm.at[pl.ds(base + i * CHUNK, CHUNK)], wsem.at[s]
            ).wait()

    return gather_kernel(x, idx3)


def _tc_linear(g, x, wa, wb, b2):
    R = 2000
    nblk = N // R

    def body(g_ref, x_ref, wa_ref, wb_ref, b_ref, o_ref):
        acc = jnp.dot(g_ref[...], wa_ref[...], preferred_element_type=jnp.float32)
        acc = acc + jnp.dot(x_ref[...], wb_ref[...], preferred_element_type=jnp.float32)
        o_ref[...] = acc + b_ref[...]

    return pl.pallas_call(
        body,
        grid=(nblk,),
        in_specs=[
            pl.BlockSpec((R, D), lambda i: (i, 0)),
            pl.BlockSpec((R, D), lambda i: (i, 0)),
            pl.BlockSpec((D, D), lambda i: (0, 0)),
            pl.BlockSpec((D, D), lambda i: (0, 0)),
            pl.BlockSpec((1, D), lambda i: (0, 0)),
        ],
        out_specs=pl.BlockSpec((R, D), lambda i: (i, 0)),
        out_shape=jax.ShapeDtypeStruct((N, D), jnp.float32),
        compiler_params=pltpu.CompilerParams(
            dimension_semantics=("arbitrary",),
        ),
    )(g, x, wa, wb, b2)


def kernel(x, idx, W2, b2):
    idx_pad = jnp.concatenate([idx, jnp.zeros((NP - N,), jnp.int32)])
    idx3 = idx_pad.reshape(NW, NCHUNK, CHUNK)
    g = _sc_gather(x, idx3)
    wa = W2[:D]
    wb = W2[D:]
    return _tc_linear(g, x, wa, wb, b2.reshape(1, D))


# speedup vs baseline: 1.0165x; 1.0064x over previous
import functools

import jax
import jax.numpy as jnp
from jax import lax
from jax.experimental import pallas as pl
from jax.experimental.pallas import tpu as pltpu
from jax.experimental.pallas import tpu_sc as plsc

N = 100000
D = 128

NC = 2
NS = 16
NW = NC * NS

CHUNK = 128
NP = 102400
CPP = NP // (NS * CHUNK)
K_FAST = 39
K_SLOW = CPP - K_FAST
NBUF = 6


def _sc_gather(x, idx3):
    mesh = plsc.VectorSubcoreMesh(
        core_axis_name="c", subcore_axis_name="s", num_cores=NC, num_subcores=NS
    )

    @functools.partial(
        pl.kernel,
        out_type=jax.ShapeDtypeStruct((NP, D), jnp.float32),
        mesh=mesh,
        scratch_types=[
            pltpu.VMEM((CPP, CHUNK), jnp.int32),
            pltpu.VMEM((NBUF, CHUNK, D), jnp.float32),
            pltpu.SemaphoreType.DMA((NBUF,)),
            pltpu.SemaphoreType.DMA((NBUF,)),
        ],
    )
    def gather_kernel(x_hbm, idx_hbm, g_hbm, idx_v, buf, gsem, wsem):
        cix = lax.axis_index("c")
        sid = lax.axis_index("s")
        loff = jnp.where(cix == 0, 0, K_FAST)
        offset = sid * CPP + loff
        count = jnp.where(cix == 0, K_FAST, K_SLOW)

        pltpu.sync_copy(idx_hbm.at[sid], idx_v)

        for j in range(NBUF - 1):
            pltpu.async_copy(x_hbm.at[idx_v.at[loff + j]], buf.at[j], gsem.at[j])

        def chunk_body(i, carry):
            slot = lax.rem(i, NBUF)
            nxt = i + NBUF - 1
            nslot = lax.rem(nxt, NBUF)

            @pl.when(nxt < count)
            def _():
                @pl.when(i >= 1)
                def _():
                    pltpu.make_async_copy(
                        buf.at[nslot],
                        g_hbm.at[pl.ds((offset + i - 1) * CHUNK, CHUNK)],
                        wsem.at[nslot],
                    ).wait()

                pltpu.async_copy(x_hbm.at[idx_v.at[loff + nxt]], buf.at[nslot], gsem.at[nslot])

            pltpu.make_async_copy(
                x_hbm.at[idx_v.at[loff + i]], buf.at[slot], gsem.at[slot]
            ).wait()
            pltpu.async_copy(
                buf.at[slot], g_hbm.at[pl.ds((offset + i) * CHUNK, CHUNK)], wsem.at[slot]
            )
            return carry

        lax.fori_loop(0, count, chunk_body, 0, unroll=False)

        def drain_body(i, carry):
            s = lax.rem(i, NBUF)
            pltpu.make_async_copy(
                buf.at[s], g_hbm.at[pl.ds((offset + i) * CHUNK, CHUNK)], wsem.at[s]
            ).wait()
            return carry

        lax.fori_loop(count - NBUF, count, drain_body, 0, unroll=False)

    return gather_kernel(x, idx3)


def _tc_linear(g, x, wa, wb, b2):
    R = 2000
    nblk = N // R

    def body(g_ref, x_ref, wa_ref, wb_ref, b_ref, o_ref):
        acc = jnp.dot(g_ref[...], wa_ref[...], preferred_element_type=jnp.float32)
        acc = acc + jnp.dot(x_ref[...], wb_ref[...], preferred_element_type=jnp.float32)
        o_ref[...] = acc + b_ref[...]

    return pl.pallas_call(
        body,
        grid=(nblk,),
        in_specs=[
            pl.BlockSpec((R, D), lambda i: (i, 0)),
            pl.BlockSpec((R, D), lambda i: (i, 0)),
            pl.BlockSpec((D, D), lambda i: (0, 0)),
            pl.BlockSpec((D, D), lambda i: (0, 0)),
            pl.BlockSpec((1, D), lambda i: (0, 0)),
        ],
        out_specs=pl.BlockSpec((R, D), lambda i: (i, 0)),
        out_shape=jax.ShapeDtypeStruct((N, D), jnp.float32),
        compiler_params=pltpu.CompilerParams(
            dimension_semantics=("arbitrary",),
        ),
    )(g, x, wa, wb, b2)


def kernel(x, idx, W2, b2):
    idx_pad = jnp.concatenate([idx, jnp.zeros((NP - N,), jnp.int32)])
    idx3 = idx_pad.reshape(NS, CPP, CHUNK)
    g = _sc_gather(x, idx3)
    wa = W2[:D]
    wb = W2[D:]
    return _tc_linear(g, x, wa, wb, b2.reshape(1, D))
